# Initial kernel scaffold; baseline (speedup 1.0000x reference)
import jax
import jax.numpy as jnp
from jax.experimental import pallas as pl


def kernel(x, neighbors, weight, bias, geo_attention):
    raise NotImplementedError("write your pallas kernel here")


# speedup vs baseline: 5.8557x; 5.8557x over previous
import functools

import jax
import jax.numpy as jnp
from jax import lax
from jax.experimental import pallas as pl
from jax.experimental.pallas import tpu as pltpu
from jax.experimental.pallas import tpu_sc as plsc

_P = 80
_NW = 32


def _transpose_body(x_ref, o_ref):
    o_ref[...] = x_ref[...].T


def _transpose(x2):
    R, N = x2.shape
    TN = 1024
    return pl.pallas_call(
        _transpose_body,
        grid=(pl.cdiv(N, TN),),
        in_specs=[pl.BlockSpec((R, TN), lambda i: (0, i))],
        out_specs=pl.BlockSpec((TN, R), lambda i: (i, 0)),
        out_shape=jax.ShapeDtypeStruct((N, R), x2.dtype),
    )(x2)


def _agg_body(xT_hbm, nbr_hbm, geo_hbm, out_hbm,
              idx_v, gv_v, w_v, rows_v, agg_v, sem):
    R = rows_v.shape[1]
    nchunk = nbr_hbm.shape[0]
    wid = lax.axis_index("s") * 2 + lax.axis_index("c")
    nper = nchunk // _NW
    extra = nchunk - nper * _NW
    start = wid * nper + jnp.minimum(wid, extra)
    count = nper + jnp.where(wid < extra, 1, 0)

    def chunk_body(j, carry):
        cid = start + j
        pltpu.sync_copy(nbr_hbm.at[cid], idx_v)
        copies = []
        for g in range(4):
            copies.append(pltpu.async_copy(
                geo_hbm.at[idx_v.at[g]], gv_v.at[g], sem))
            copies.append(pltpu.async_copy(
                xT_hbm.at[idx_v.at[g]],
                rows_v.at[pl.ds(g * _P, _P)], sem))
        for c in copies:
            c.wait()

        for grp in range(_P // 16):
            sl = pl.ds(grp * 16, 16)
            a = [gv_v[k, sl] for k in range(4)]
            m = jnp.maximum(jnp.maximum(a[0], a[1]), jnp.maximum(a[2], a[3]))
            e = [jnp.exp(ak - m) for ak in a]
            s = e[0] + e[1] + e[2] + e[3]
            r = 1.0 / s
            for k in range(4):
                w_v[k, sl] = e[k] * r

        def point_body(p, carry2):
            w = [w_v[k, pl.ds(p, 16)][0] for k in range(4)]
            for c in range(R // 16):
                sl = pl.ds(c * 16, 16)
                acc = (w[0] * rows_v[p, sl]
                       + w[1] * rows_v[_P + p, sl]
                       + w[2] * rows_v[2 * _P + p, sl]
                       + w[3] * rows_v[3 * _P + p, sl])
                agg_v[p, sl] = acc
            return carry2

        lax.fori_loop(0, _P, point_body, 0)
        pltpu.sync_copy(agg_v, out_hbm.at[pl.ds(cid * _P, _P)])
        return carry

    lax.fori_loop(0, count, chunk_body, 0)


def _aggregate(xT, nbr_c, geo):
    N, R = xT.shape
    mesh = plsc.VectorSubcoreMesh(core_axis_name="c", subcore_axis_name="s")
    f = pl.kernel(
        _agg_body,
        out_type=jax.ShapeDtypeStruct((N, R), jnp.float32),
        mesh=mesh,
        scratch_types=[
            pltpu.VMEM((4, _P), jnp.int32),
            pltpu.VMEM((4, _P), jnp.float32),
            pltpu.VMEM((4, _P + 16), jnp.float32),
            pltpu.VMEM((4 * _P, R), jnp.float32),
            pltpu.VMEM((_P, R), jnp.float32),
            pltpu.SemaphoreType.DMA,
        ],
    )
    return f(xT, nbr_c, geo)


def _transpose_wide_body(a_ref, o_ref):
    o_ref[...] = a_ref[...].T


def _transpose_wide(a):
    M, R = a.shape
    TM = 1024
    return pl.pallas_call(
        _transpose_wide_body,
        grid=(pl.cdiv(M, TM),),
        in_specs=[pl.BlockSpec((TM, R), lambda i: (i, 0))],
        out_specs=pl.BlockSpec((R, TM), lambda i: (0, i)),
        out_shape=jax.ShapeDtypeStruct((R, M), a.dtype),
    )(a)


def _linear_body(a_ref, w_ref, b_ref, o_ref):
    z = a_ref[...]
    w = w_ref[...]
    bias = b_ref[...]
    _, CO, CI = w.shape
    R, NT = z.shape
    nb = R // CI
    lane_mod = lax.broadcasted_iota(jnp.int32, (CI, NT), 1) % 4
    outs = []
    for bb in range(nb):
        zb = z[bb * CI:(bb + 1) * CI, :]
        acc = None
        for k in range(4):
            zk = jnp.where(lane_mod == k, zb, 0.0).astype(jnp.bfloat16)
            ok = lax.dot_general(
                w[k].astype(jnp.bfloat16), zk, (((1,), (0,)), ((), ())),
                preferred_element_type=jnp.float32)
            acc = ok if acc is None else acc + ok
        outs.append(acc + bias[:, None])
    o_ref[...] = jnp.stack(outs, axis=0)


def _linear(agg2, weight_r, bias):
    R, N = agg2.shape
    _, CO, CI = weight_r.shape
    nb = R // CI
    NT = 4096
    return pl.pallas_call(
        _linear_body,
        grid=(pl.cdiv(N, NT),),
        in_specs=[
            pl.BlockSpec((R, NT), lambda q: (0, q)),
            pl.BlockSpec((4, CO, CI), lambda q: (0, 0, 0)),
            pl.BlockSpec((CO,), lambda q: (0,)),
        ],
        out_specs=pl.BlockSpec((nb, CO, NT), lambda q: (0, 0, q)),
        out_shape=jax.ShapeDtypeStruct((nb, CO, N), jnp.float32),
    )(agg2, weight_r, bias)


def kernel(x, neighbors, weight, bias, geo_attention):
    B, C, N = x.shape
    R = B * C
    x2 = x.reshape(R, N)
    xT = _transpose(x2)
    nchunk = N // _P
    nbr_c = jnp.transpose(neighbors.reshape(nchunk, _P, 4), (0, 2, 1))
    aggT = _aggregate(xT, nbr_c, geo_attention)
    agg2 = _transpose_wide(aggT)
    weight_r = jnp.transpose(weight, (2, 0, 1))
    return _linear(agg2, weight_r, bias)


# speedup vs baseline: 6.5901x; 1.1254x over previous
import functools

import jax
import jax.numpy as jnp
from jax import lax
from jax.experimental import pallas as pl
from jax.experimental.pallas import tpu as pltpu
from jax.experimental.pallas import tpu_sc as plsc

_P = 80
_NW = 32


def _transpose_body(x_ref, o_ref):
    x = x_ref[...]
    B_ = x.shape[0]
    o_ref[...] = jnp.concatenate([x[b].T for b in range(B_)], axis=1)


def _transpose(x):
    B_, C, N = x.shape
    TN = 1024
    return pl.pallas_call(
        _transpose_body,
        grid=(pl.cdiv(N, TN),),
        in_specs=[pl.BlockSpec((B_, C, TN), lambda i: (0, 0, i))],
        out_specs=pl.BlockSpec((TN, B_ * C), lambda i: (i, 0)),
        out_shape=jax.ShapeDtypeStruct((N, B_ * C), x.dtype),
    )(x)


def _agg_body(xT_hbm, nbr_hbm, geo_hbm, out_hbm,
              idx_v, gv_v, w_v, rows_v, agg_v, sem):
    R = rows_v.shape[1]
    nchunk = nbr_hbm.shape[0]
    wid = lax.axis_index("s") * 2 + lax.axis_index("c")
    nper = nchunk // _NW
    extra = nchunk - nper * _NW
    start = wid * nper + jnp.minimum(wid, extra)
    count = nper + jnp.where(wid < extra, 1, 0)

    def chunk_body(j, carry):
        cid = start + j
        pltpu.sync_copy(nbr_hbm.at[cid], idx_v)
        copies = []
        for g in range(4):
            copies.append(pltpu.async_copy(
                geo_hbm.at[idx_v.at[g]], gv_v.at[g], sem))
            copies.append(pltpu.async_copy(
                xT_hbm.at[idx_v.at[g]],
                rows_v.at[pl.ds(g * _P, _P)], sem))
        for c in copies:
            c.wait()

        for grp in range(_P // 16):
            sl = pl.ds(grp * 16, 16)
            a = [gv_v[k, sl] for k in range(4)]
            m = jnp.maximum(jnp.maximum(a[0], a[1]), jnp.maximum(a[2], a[3]))
            e = [jnp.exp(ak - m) for ak in a]
            s = e[0] + e[1] + e[2] + e[3]
            r = 1.0 / s
            for k in range(4):
                w_v[k, sl] = e[k] * r

        def point_body(p, carry2):
            w = [w_v[k, pl.ds(p, 16)][0] for k in range(4)]
            for c in range(R // 16):
                sl = pl.ds(c * 16, 16)
                acc = (w[0] * rows_v[p, sl]
                       + w[1] * rows_v[_P + p, sl]
                       + w[2] * rows_v[2 * _P + p, sl]
                       + w[3] * rows_v[3 * _P + p, sl])
                agg_v[p, sl] = acc
            return carry2

        lax.fori_loop(0, _P, point_body, 0)
        pltpu.sync_copy(agg_v, out_hbm.at[pl.ds(cid * _P, _P)])
        return carry

    lax.fori_loop(0, count, chunk_body, 0)


def _aggregate(xT, nbr_c, geo):
    N, R = xT.shape
    mesh = plsc.VectorSubcoreMesh(core_axis_name="c", subcore_axis_name="s")
    f = pl.kernel(
        _agg_body,
        out_type=jax.ShapeDtypeStruct((N, R), jnp.float32),
        mesh=mesh,
        scratch_types=[
            pltpu.VMEM((4, _P), jnp.int32),
            pltpu.VMEM((4, _P), jnp.float32),
            pltpu.VMEM((4, _P + 16), jnp.float32),
            pltpu.VMEM((4 * _P, R), jnp.float32),
            pltpu.VMEM((_P, R), jnp.float32),
            pltpu.SemaphoreType.DMA,
        ],
    )
    return f(xT, nbr_c, geo)


def _linear_body(a_ref, w_ref, b_ref, o_ref):
    z = a_ref[...].T
    w = w_ref[...]
    bias = b_ref[...]
    _, CO, CI = w.shape
    R, NT = z.shape
    nb = R // CI
    lane_mod = lax.broadcasted_iota(jnp.int32, (CI, NT), 1) % 4
    outs = []
    for bb in range(nb):
        zb = z[bb * CI:(bb + 1) * CI, :]
        acc = None
        for k in range(4):
            zk = jnp.where(lane_mod == k, zb, 0.0).astype(jnp.bfloat16)
            ok = lax.dot_general(
                w[k].astype(jnp.bfloat16), zk, (((1,), (0,)), ((), ())),
                preferred_element_type=jnp.float32)
            acc = ok if acc is None else acc + ok
        outs.append(acc + bias[:, None])
    o_ref[...] = jnp.stack(outs, axis=0)


def _linear(aggT, weight_r, bias):
    N, R = aggT.shape
    _, CO, CI = weight_r.shape
    nb = R // CI
    NT = 4096
    return pl.pallas_call(
        _linear_body,
        grid=(pl.cdiv(N, NT),),
        in_specs=[
            pl.BlockSpec((NT, R), lambda q: (q, 0)),
            pl.BlockSpec((4, CO, CI), lambda q: (0, 0, 0)),
            pl.BlockSpec((CO,), lambda q: (0,)),
        ],
        out_specs=pl.BlockSpec((nb, CO, NT), lambda q: (0, 0, q)),
        out_shape=jax.ShapeDtypeStruct((nb, CO, N), jnp.float32),
    )(aggT, weight_r, bias)


def kernel(x, neighbors, weight, bias, geo_attention):
    B, C, N = x.shape
    xT = _transpose(x)
    nchunk = N // _P
    nbr_c = jnp.transpose(neighbors.reshape(nchunk, _P, 4), (0, 2, 1))
    aggT = _aggregate(xT, nbr_c, geo_attention)
    weight_r = jnp.transpose(weight, (2, 0, 1))
    return _linear(aggT, weight_r, bias)


# speedup vs baseline: 7.5396x; 1.1441x over previous
import functools

import jax
import jax.numpy as jnp
from jax import lax
from jax.experimental import pallas as pl
from jax.experimental.pallas import tpu as pltpu
from jax.experimental.pallas import tpu_sc as plsc

_P = 40
_GW = 48
_NW = 32


def _transpose_body(x_ref, o_ref):
    x = x_ref[...]
    B_ = x.shape[0]
    o_ref[...] = jnp.concatenate([x[b].T for b in range(B_)], axis=1)


def _transpose(x):
    B_, C, N = x.shape
    TN = 1024
    return pl.pallas_call(
        _transpose_body,
        grid=(pl.cdiv(N, TN),),
        in_specs=[pl.BlockSpec((B_, C, TN), lambda i: (0, 0, i))],
        out_specs=pl.BlockSpec((TN, B_ * C), lambda i: (i, 0)),
        out_shape=jax.ShapeDtypeStruct((N, B_ * C), x.dtype),
    )(x)


def _agg_body(xT_hbm, nbr_hbm, geo_hbm, out_hbm,
              idx0, idx1, gv0, gv1, wv0, wv1, rows0, rows1, agg0, agg1,
              sg0, sg1, ss0, ss1):
    R = rows0.shape[1]
    N = nbr_hbm.shape[0] // 4
    nchunk = N // _P
    per_worker = (nchunk + _NW - 1) // _NW
    pairs = per_worker // 2
    wid = lax.axis_index("s") * 2 + lax.axis_index("c")
    base = wid * per_worker

    def cid_of(i):
        return jnp.minimum(base + i, nchunk - 1)

    def idx_copies(cid, idxb, semg):
        return [pltpu.make_async_copy(
            nbr_hbm.at[pl.ds(g * N + cid * _P, _P)], idxb.at[g], semg)
            for g in range(4)]

    def gather_copies(cid, idxb, gvb, rowsb, semg):
        copies = []
        for g in range(4):
            copies.append(pltpu.make_async_copy(
                geo_hbm.at[idxb.at[g]], gvb.at[pl.ds(g * _GW, _P)], semg))
            copies.append(pltpu.make_async_copy(
                xT_hbm.at[idxb.at[g]], rowsb.at[pl.ds(g * _P, _P)], semg))
        return copies

    def issue(cid, idxb, gvb, rowsb, semg):
        ics = idx_copies(cid, idxb, semg)
        for c in ics:
            c.start()
        for c in ics:
            c.wait()
        for c in gather_copies(cid, idxb, gvb, rowsb, semg):
            c.start()

    def wait_gathers(cid, idxb, gvb, rowsb, semg):
        for c in gather_copies(cid, idxb, gvb, rowsb, semg):
            c.wait()

    def compute(gvb, wvb, rowsb, aggb):
        for grp in range(3):
            a = [gvb[pl.ds(k * _GW + grp * 16, 16)] for k in range(4)]
            m = jnp.maximum(jnp.maximum(a[0], a[1]), jnp.maximum(a[2], a[3]))
            e = [jnp.exp(ak - m) for ak in a]
            r = 1.0 / (e[0] + e[1] + e[2] + e[3])
            for k in range(4):
                wvb[pl.ds(k * _GW + grp * 16, 16)] = e[k] * r

        def point_body(p, carry2):
            w = [wvb[pl.ds(k * _GW + p, 16)][0] for k in range(4)]
            for c in range(R // 16):
                sl = pl.ds(c * 16, 16)
                acc = (w[0] * rowsb[p, sl]
                       + w[1] * rowsb[_P + p, sl]
                       + w[2] * rowsb[2 * _P + p, sl]
                       + w[3] * rowsb[3 * _P + p, sl])
                aggb[p, sl] = acc
            return carry2

        lax.fori_loop(0, _P, point_body, 0)

    def store(aggb, cid, sems):
        pltpu.make_async_copy(
            aggb, out_hbm.at[pl.ds(cid * _P, _P)], sems).start()

    def wait_store(aggb, cid, sems):
        pltpu.make_async_copy(
            aggb, out_hbm.at[pl.ds(cid * _P, _P)], sems).wait()

    issue(cid_of(0), idx0, gv0, rows0, sg0)

    def pair_body(t, carry):
        c0 = cid_of(2 * t)
        c1 = cid_of(2 * t + 1)
        c2 = cid_of(2 * t + 2)
        issue(c1, idx1, gv1, rows1, sg1)
        wait_gathers(c0, idx0, gv0, rows0, sg0)

        @pl.when(t > 0)
        def _():
            wait_store(agg0, c0, ss0)

        compute(gv0, wv0, rows0, agg0)
        store(agg0, c0, ss0)
        issue(c2, idx0, gv0, rows0, sg0)
        wait_gathers(c1, idx1, gv1, rows1, sg1)

        @pl.when(t > 0)
        def _():
            wait_store(agg1, c1, ss1)

        compute(gv1, wv1, rows1, agg1)
        store(agg1, c1, ss1)
        return carry

    lax.fori_loop(0, pairs, pair_body, 0)
    wait_gathers(cid_of(0), idx0, gv0, rows0, sg0)
    wait_store(agg0, cid_of(0), ss0)
    wait_store(agg1, cid_of(0), ss1)


def _aggregate(xT, nbr_k, geo):
    N, R = xT.shape
    mesh = plsc.VectorSubcoreMesh(core_axis_name="c", subcore_axis_name="s")
    f = pl.kernel(
        _agg_body,
        out_type=jax.ShapeDtypeStruct((N, R), jnp.float32),
        mesh=mesh,
        scratch_types=(
            [pltpu.VMEM((4, _P), jnp.int32)] * 2
            + [pltpu.VMEM((4 * _GW,), jnp.float32)] * 2
            + [pltpu.VMEM((4 * _GW + 16,), jnp.float32)] * 2
            + [pltpu.VMEM((4 * _P, R), jnp.float32)] * 2
            + [pltpu.VMEM((_P, R), jnp.float32)] * 2
            + [pltpu.SemaphoreType.DMA] * 4
        ),
    )
    return f(xT, nbr_k, geo)


def _linear_body(a_ref, w_ref, b_ref, o_ref):
    z = a_ref[...].T
    w = w_ref[...]
    bias = b_ref[...]
    _, CO, CI = w.shape
    R, NT = z.shape
    nb = R // CI
    lane_mod = lax.broadcasted_iota(jnp.int32, (CI, NT), 1) % 4
    outs = []
    for bb in range(nb):
        zb = z[bb * CI:(bb + 1) * CI, :]
        acc = None
        for k in range(4):
            zk = jnp.where(lane_mod == k, zb, 0.0).astype(jnp.bfloat16)
            ok = lax.dot_general(
                w[k].astype(jnp.bfloat16), zk, (((1,), (0,)), ((), ())),
                preferred_element_type=jnp.float32)
            acc = ok if acc is None else acc + ok
        outs.append(acc + bias[:, None])
    o_ref[...] = jnp.stack(outs, axis=0)


def _linear(aggT, weight_r, bias):
    N, R = aggT.shape
    _, CO, CI = weight_r.shape
    nb = R // CI
    NT = 4096
    return pl.pallas_call(
        _linear_body,
        grid=(pl.cdiv(N, NT),),
        in_specs=[
            pl.BlockSpec((NT, R), lambda q: (q, 0)),
            pl.BlockSpec((4, CO, CI), lambda q: (0, 0, 0)),
            pl.BlockSpec((CO,), lambda q: (0,)),
        ],
        out_specs=pl.BlockSpec((nb, CO, NT), lambda q: (0, 0, q)),
        out_shape=jax.ShapeDtypeStruct((nb, CO, N), jnp.float32),
    )(aggT, weight_r, bias)


def kernel(x, neighbors, weight, bias, geo_attention):
    B, C, N = x.shape
    xT = _transpose(x)
    nbr_k = neighbors.T.reshape(-1)
    aggT = _aggregate(xT, nbr_k, geo_attention)
    weight_r = jnp.transpose(weight, (2, 0, 1))
    return _linear(aggT, weight_r, bias)


# speedup vs baseline: 15.6158x; 2.0712x over previous
import jax
import jax.numpy as jnp
from jax import lax
from jax.experimental import pallas as pl
from jax.experimental.pallas import tpu as pltpu
from jax.experimental.pallas import tpu_sc as plsc

_P = 40
_GW = 48
_NW = 32


def _agg_body(x_hbm, nbr_hbm, geo_hbm, out_hbm,
              idx0, idx1, gv0, gv1, wv0, wv1, rows0, rows1, agg0, agg1,
              sg0, sg1, ss0, ss1):
    NB = x_hbm.shape[0]
    C = x_hbm.shape[2]
    N = x_hbm.shape[1]
    nchunk = N // _P
    per_worker = (nchunk + _NW - 1) // _NW
    pairs = per_worker // 2
    wid = lax.axis_index("s") * 2 + lax.axis_index("c")
    base = wid * per_worker

    def cid_of(i):
        return jnp.minimum(base + i, nchunk - 1)

    def idx_copies(cid, idxb, semg):
        return [pltpu.make_async_copy(
            nbr_hbm.at[pl.ds(g * N + cid * _P, _P)], idxb.at[g], semg)
            for g in range(4)]

    def gather_copies(cid, idxb, gvb, rowsb, semg):
        copies = []
        for g in range(4):
            copies.append(pltpu.make_async_copy(
                geo_hbm.at[idxb.at[g]], gvb.at[pl.ds(g * _GW, _P)], semg))
            for b in range(NB):
                copies.append(pltpu.make_async_copy(
                    x_hbm.at[b].at[idxb.at[g]],
                    rowsb.at[b, pl.ds(g * _P, _P)], semg))
        return copies

    def issue(cid, idxb, gvb, rowsb, semg):
        ics = idx_copies(cid, idxb, semg)
        for c in ics:
            c.start()
        for c in ics:
            c.wait()
        for c in gather_copies(cid, idxb, gvb, rowsb, semg):
            c.start()

    def wait_gathers(cid, idxb, gvb, rowsb, semg):
        for c in gather_copies(cid, idxb, gvb, rowsb, semg):
            c.wait()

    def compute(gvb, wvb, rowsb, aggb):
        for grp in range(3):
            a = [gvb[pl.ds(k * _GW + grp * 16, 16)] for k in range(4)]
            m = jnp.maximum(jnp.maximum(a[0], a[1]), jnp.maximum(a[2], a[3]))
            e = [jnp.exp(ak - m) for ak in a]
            r = 1.0 / (e[0] + e[1] + e[2] + e[3])
            for k in range(4):
                wvb[pl.ds(k * _GW + grp * 16, 16)] = e[k] * r

        def point_body(p, carry2):
            w = [wvb[pl.ds(k * _GW + p, 16)][0] for k in range(4)]
            for b in range(NB):
                for c in range(C // 16):
                    sl = pl.ds(c * 16, 16)
                    acc = (w[0] * rowsb[b, p, sl]
                           + w[1] * rowsb[b, _P + p, sl]
                           + w[2] * rowsb[b, 2 * _P + p, sl]
                           + w[3] * rowsb[b, 3 * _P + p, sl])
                    aggb[b, p, sl] = acc
            return carry2

        lax.fori_loop(0, _P, point_body, 0)

    def store_copies(aggb, cid, sems):
        return [pltpu.make_async_copy(
            aggb.at[b], out_hbm.at[b, pl.ds(cid * _P, _P)], sems)
            for b in range(NB)]

    def store(aggb, cid, sems):
        for c in store_copies(aggb, cid, sems):
            c.start()

    def wait_store(aggb, cid, sems):
        for c in store_copies(aggb, cid, sems):
            c.wait()

    issue(cid_of(0), idx0, gv0, rows0, sg0)

    def pair_body(t, carry):
        c0 = cid_of(2 * t)
        c1 = cid_of(2 * t + 1)
        c2 = cid_of(2 * t + 2)
        issue(c1, idx1, gv1, rows1, sg1)
        wait_gathers(c0, idx0, gv0, rows0, sg0)

        @pl.when(t > 0)
        def _():
            wait_store(agg0, c0, ss0)

        compute(gv0, wv0, rows0, agg0)
        store(agg0, c0, ss0)
        issue(c2, idx0, gv0, rows0, sg0)
        wait_gathers(c1, idx1, gv1, rows1, sg1)

        @pl.when(t > 0)
        def _():
            wait_store(agg1, c1, ss1)

        compute(gv1, wv1, rows1, agg1)
        store(agg1, c1, ss1)
        return carry

    lax.fori_loop(0, pairs, pair_body, 0)
    wait_gathers(cid_of(0), idx0, gv0, rows0, sg0)
    wait_store(agg0, cid_of(0), ss0)
    wait_store(agg1, cid_of(0), ss1)


def _aggregate(x_bnc, nbr_k, geo):
    NB, N, C = x_bnc.shape
    mesh = plsc.VectorSubcoreMesh(core_axis_name="c", subcore_axis_name="s")
    f = pl.kernel(
        _agg_body,
        out_type=jax.ShapeDtypeStruct((NB, N, C), jnp.float32),
        mesh=mesh,
        scratch_types=(
            [pltpu.VMEM((4, _P), jnp.int32)] * 2
            + [pltpu.VMEM((4 * _GW,), jnp.float32)] * 2
            + [pltpu.VMEM((4 * _GW + 16,), jnp.float32)] * 2
            + [pltpu.VMEM((NB, 4 * _P, C), jnp.float32)] * 2
            + [pltpu.VMEM((NB, _P, C), jnp.float32)] * 2
            + [pltpu.SemaphoreType.DMA] * 4
        ),
    )
    return f(x_bnc, nbr_k, geo)


def _linear_body(a_ref, w_ref, b_ref, o_ref):
    a = a_ref[...]
    w = w_ref[...]
    bias = b_ref[...]
    NB, NT, CI = a.shape
    row_mod = lax.broadcasted_iota(jnp.int32, (NT, CI), 0) % 4
    outs = []
    for b in range(NB):
        acc = None
        for k in range(4):
            zk = jnp.where(row_mod == k, a[b], 0.0).astype(jnp.bfloat16)
            ok = lax.dot_general(
                zk, w[k].astype(jnp.bfloat16), (((1,), (0,)), ((), ())),
                preferred_element_type=jnp.float32)
            acc = ok if acc is None else acc + ok
        outs.append(acc + bias[None, :])
    o_ref[...] = jnp.stack(outs, axis=0)


def _linear(aggb, weight_r, bias):
    NB, N, CI = aggb.shape
    CO = weight_r.shape[2]
    NT = 4096
    return pl.pallas_call(
        _linear_body,
        grid=(pl.cdiv(N, NT),),
        in_specs=[
            pl.BlockSpec((NB, NT, CI), lambda q: (0, q, 0)),
            pl.BlockSpec((4, CI, CO), lambda q: (0, 0, 0)),
            pl.BlockSpec((CO,), lambda q: (0,)),
        ],
        out_specs=pl.BlockSpec((NB, NT, CO), lambda q: (0, q, 0)),
        out_shape=jax.ShapeDtypeStruct((NB, N, CO), jnp.float32),
    )(aggb, weight_r, bias)


def kernel(x, neighbors, weight, bias, geo_attention):
    B, C, N = x.shape
    x_bnc = jnp.transpose(x, (0, 2, 1))
    nbr_k = neighbors.T.reshape(-1)
    aggb = _aggregate(x_bnc, nbr_k, geo_attention)
    weight_r = jnp.transpose(weight, (2, 1, 0))
    out_bnc = _linear(aggb, weight_r, bias)
    return jnp.transpose(out_bnc, (0, 2, 1))
